# baseline (speedup 1.0000x reference)
import jax
import jax.numpy as jnp
from jax.experimental import pallas as pl
from jax.experimental.pallas import tpu as pltpu

_L = 512
_LS = 256


def _body(x_ref, wb_ref, wc_ref, wd_ref, tri_ref, nu_ref, th_ref, gl_ref,
          y_ref, h_ref, wr_s, wi_s, vr_s, vi_s, lam_s):
    n = nu_ref.shape[1]
    ls = tri_ref.shape[0]
    nsub = _L // ls
    b_idx = pl.program_id(0)
    t_idx = pl.program_id(1)

    @pl.when((b_idx == 0) & (t_idx == 0))
    def _():
        nu = jnp.exp(nu_ref[...])
        th = jnp.exp(th_ref[...])
        t = jax.lax.broadcasted_iota(jnp.int32, (ls, n), 0).astype(jnp.float32)
        ang = t * th
        ct, st = jnp.cos(ang), jnp.sin(ang)
        tnu = t * nu
        mag_pos = jnp.exp(-tnu)
        mag_neg_g = jnp.exp(tnu + gl_ref[...])
        wr_s[...] = mag_neg_g * ct
        wi_s[...] = -(mag_neg_g * st)
        vr_s[...] = mag_pos * ct
        vi_s[...] = mag_pos * st
        lam_s[...] = jnp.concatenate(
            [jnp.exp(-nu) * jnp.cos(th), jnp.exp(-nu) * jnp.sin(th)], axis=1)

    @pl.when(t_idx == 0)
    def _():
        h_ref[...] = jnp.zeros_like(h_ref)

    xb = x_ref[0].astype(jnp.bfloat16)
    z = jnp.dot(xb, wb_ref[...], preferred_element_type=jnp.float32)
    wr, wi = wr_s[...], wi_s[...]
    vr, vi = vr_s[...], vi_s[...]
    lam = lam_s[...]
    lr, li = lam[:, :n], lam[:, n:]
    cs = []
    for k in range(nsub):
        zk = z[k * ls:(k + 1) * ls]
        zr, zi = zk[:, :n], zk[:, n:]
        bp = jnp.concatenate([wr * zr - wi * zi, wi * zr + wr * zi], axis=1)
        cs.append(jnp.dot(tri_ref[...], bp.astype(jnp.bfloat16),
                          preferred_element_type=jnp.float32))
    h = h_ref[...]
    hr, hi = h[:, :n], h[:, n:]
    srs, sis = [], []
    for k in range(nsub):
        cr = cs[k][:, :n] + (lr * hr - li * hi)
        ci = cs[k][:, n:] + (li * hr + lr * hi)
        sr = vr * cr - vi * ci
        si = vi * cr + vr * ci
        srs.append(sr.astype(jnp.bfloat16))
        sis.append(si.astype(jnp.bfloat16))
        hr, hi = sr[ls - 1:ls, :], si[ls - 1:ls, :]
    h_ref[...] = jnp.concatenate([hr, hi], axis=1)
    yd = jnp.dot(xb, wd_ref[...], preferred_element_type=jnp.float32)
    sx = jnp.concatenate([jnp.concatenate(srs, axis=0),
                          jnp.concatenate(sis, axis=0)], axis=1)
    y_ref[0] = jnp.dot(sx, wc_ref[...], preferred_element_type=jnp.float32) + yd


def kernel(x, nu_log, theta_log, gamma_log, B_re, B_im, C_re, C_im, D):
    b_sz, t_len, d_in = x.shape
    d_out = D.shape[0]
    n = nu_log.shape[0]
    L, LS = _L, _LS
    n_chunks = t_len // L

    Wb = jnp.concatenate([B_re.T, B_im.T], axis=1).astype(jnp.bfloat16)
    Wc = jnp.concatenate([C_re.T, -C_im.T], axis=0).astype(jnp.bfloat16)
    Wd = D.T.astype(jnp.bfloat16)
    tri = jnp.tril(jnp.ones((LS, LS), jnp.float32)).astype(jnp.bfloat16)

    const = lambda *_: (0, 0)
    grid = (b_sz, n_chunks)
    y = pl.pallas_call(
        _body,
        out_shape=jax.ShapeDtypeStruct((b_sz, t_len, d_out), jnp.float32),
        grid=grid,
        in_specs=[
            pl.BlockSpec((1, L, d_in), lambda b, tc: (b, tc, 0)),
            pl.BlockSpec((d_in, 2 * n), const),
            pl.BlockSpec((2 * n, d_out), const),
            pl.BlockSpec((d_in, d_out), const),
            pl.BlockSpec((LS, LS), const),
            pl.BlockSpec((1, n), const),
            pl.BlockSpec((1, n), const),
            pl.BlockSpec((1, n), const),
        ],
        out_specs=pl.BlockSpec((1, L, d_out), lambda b, tc: (b, tc, 0)),
        scratch_shapes=[
            pltpu.VMEM((1, 2 * n), jnp.float32),
            pltpu.VMEM((LS, n), jnp.float32),
            pltpu.VMEM((LS, n), jnp.float32),
            pltpu.VMEM((LS, n), jnp.float32),
            pltpu.VMEM((LS, n), jnp.float32),
            pltpu.VMEM((1, 2 * n), jnp.float32),
        ],
        compiler_params=pltpu.CompilerParams(
            dimension_semantics=("arbitrary", "arbitrary"),
            vmem_limit_bytes=56 * 1024 * 1024,
        ),
        name="lru_fused",
    )(x, Wb, Wc, Wd, tri, nu_log[None, :], theta_log[None, :],
      gamma_log[None, :])
    return y


# speedup vs baseline: 1.0292x; 1.0292x over previous
import jax
import jax.numpy as jnp
from jax.experimental import pallas as pl
from jax.experimental.pallas import tpu as pltpu

_L = 512
_LS = 256


def _body(x_ref, bre_ref, bim_ref, cre_ref, cim_ref, d_ref, tri_ref,
          nu_ref, th_ref, gl_ref, y_ref,
          h_ref, wr_s, wi_s, vr_s, vi_s, lam_s, wb_s, wc_s, wd_s):
    n = nu_ref.shape[1]
    ls = tri_ref.shape[0]
    nsub = _L // ls
    b_idx = pl.program_id(0)
    t_idx = pl.program_id(1)

    @pl.when((b_idx == 0) & (t_idx == 0))
    def _():
        nu = jnp.exp(nu_ref[...])
        th = jnp.exp(th_ref[...])
        t = jax.lax.broadcasted_iota(jnp.int32, (ls, n), 0).astype(jnp.float32)
        ang = t * th
        ct, st = jnp.cos(ang), jnp.sin(ang)
        tnu = t * nu
        mag_pos = jnp.exp(-tnu)
        mag_neg_g = jnp.exp(tnu + gl_ref[...])
        wr_s[...] = mag_neg_g * ct
        wi_s[...] = -(mag_neg_g * st)
        vr_s[...] = mag_pos * ct
        vi_s[...] = mag_pos * st
        lam_s[...] = jnp.concatenate(
            [jnp.exp(-nu) * jnp.cos(th), jnp.exp(-nu) * jnp.sin(th)], axis=1)

        def prep(buf, sem):
            cp = pltpu.make_async_copy(bre_ref, buf.at[:n, :], sem)
            cp.start()
            cp.wait()
            wb_s[:, :n] = jnp.transpose(buf[:n, :]).astype(jnp.bfloat16)
            cp = pltpu.make_async_copy(bim_ref, buf.at[:n, :], sem)
            cp.start()
            cp.wait()
            wb_s[:, n:] = jnp.transpose(buf[:n, :]).astype(jnp.bfloat16)
            cp = pltpu.make_async_copy(cre_ref, buf.at[:, :n], sem)
            cp.start()
            cp.wait()
            wc_s[:n, :] = jnp.transpose(buf[:, :n]).astype(jnp.bfloat16)
            cp = pltpu.make_async_copy(cim_ref, buf.at[:, :n], sem)
            cp.start()
            cp.wait()
            wc_s[n:, :] = (-jnp.transpose(buf[:, :n])).astype(jnp.bfloat16)
            cp = pltpu.make_async_copy(d_ref, buf, sem)
            cp.start()
            cp.wait()
            wd_s[...] = jnp.transpose(buf[...]).astype(jnp.bfloat16)

        pl.run_scoped(prep,
                      pltpu.VMEM(d_ref.shape, jnp.float32),
                      pltpu.SemaphoreType.DMA)

    @pl.when(t_idx == 0)
    def _():
        h_ref[...] = jnp.zeros_like(h_ref)

    xb = x_ref[0].astype(jnp.bfloat16)
    z = jnp.dot(xb, wb_s[...], preferred_element_type=jnp.float32)
    wr, wi = wr_s[...], wi_s[...]
    vr, vi = vr_s[...], vi_s[...]
    lam = lam_s[...]
    lr, li = lam[:, :n], lam[:, n:]
    cs = []
    for k in range(nsub):
        zk = z[k * ls:(k + 1) * ls]
        zr, zi = zk[:, :n], zk[:, n:]
        bp = jnp.concatenate([wr * zr - wi * zi, wi * zr + wr * zi], axis=1)
        cs.append(jnp.dot(tri_ref[...], bp.astype(jnp.bfloat16),
                          preferred_element_type=jnp.float32))
    h = h_ref[...]
    hr, hi = h[:, :n], h[:, n:]
    srs, sis = [], []
    for k in range(nsub):
        cr = cs[k][:, :n] + (lr * hr - li * hi)
        ci = cs[k][:, n:] + (li * hr + lr * hi)
        sr = vr * cr - vi * ci
        si = vi * cr + vr * ci
        srs.append(sr.astype(jnp.bfloat16))
        sis.append(si.astype(jnp.bfloat16))
        hr, hi = sr[ls - 1:ls, :], si[ls - 1:ls, :]
    h_ref[...] = jnp.concatenate([hr, hi], axis=1)
    yd = jnp.dot(xb, wd_s[...], preferred_element_type=jnp.float32)
    sx = jnp.concatenate([jnp.concatenate(srs, axis=0),
                          jnp.concatenate(sis, axis=0)], axis=1)
    y_ref[0] = jnp.dot(sx, wc_s[...], preferred_element_type=jnp.float32) + yd


def kernel(x, nu_log, theta_log, gamma_log, B_re, B_im, C_re, C_im, D):
    b_sz, t_len, d_in = x.shape
    d_out = D.shape[0]
    n = nu_log.shape[0]
    L, LS = _L, _LS
    n_chunks = t_len // L

    tri = jnp.tril(jnp.ones((LS, LS), jnp.float32)).astype(jnp.bfloat16)

    const = lambda *_: (0, 0)
    any_spec = pl.BlockSpec(memory_space=pl.ANY)
    grid = (b_sz, n_chunks)
    y = pl.pallas_call(
        _body,
        out_shape=jax.ShapeDtypeStruct((b_sz, t_len, d_out), jnp.float32),
        grid=grid,
        in_specs=[
            pl.BlockSpec((1, L, d_in), lambda b, tc: (b, tc, 0)),
            any_spec,
            any_spec,
            any_spec,
            any_spec,
            any_spec,
            pl.BlockSpec((LS, LS), const),
            pl.BlockSpec((1, n), const),
            pl.BlockSpec((1, n), const),
            pl.BlockSpec((1, n), const),
        ],
        out_specs=pl.BlockSpec((1, L, d_out), lambda b, tc: (b, tc, 0)),
        scratch_shapes=[
            pltpu.VMEM((1, 2 * n), jnp.float32),
            pltpu.VMEM((LS, n), jnp.float32),
            pltpu.VMEM((LS, n), jnp.float32),
            pltpu.VMEM((LS, n), jnp.float32),
            pltpu.VMEM((LS, n), jnp.float32),
            pltpu.VMEM((1, 2 * n), jnp.float32),
            pltpu.VMEM((d_in, 2 * n), jnp.bfloat16),
            pltpu.VMEM((2 * n, d_out), jnp.bfloat16),
            pltpu.VMEM((d_in, d_out), jnp.bfloat16),
        ],
        compiler_params=pltpu.CompilerParams(
            dimension_semantics=("arbitrary", "arbitrary"),
            vmem_limit_bytes=56 * 1024 * 1024,
        ),
        name="lru_fused",
    )(x, B_re, B_im, C_re, C_im, D, tri, nu_log[None, :], theta_log[None, :],
      gamma_log[None, :])
    return y
